# baseline (speedup 1.0000x reference)
import jax
import jax.numpy as jnp
from jax import lax
from jax.experimental import pallas as pl
from jax.experimental.pallas import tpu as pltpu

_B, _S, _D = 4, 2048, 1024
_H = 16
_HD = _D // _H
_MASK = -1000000.0
_LOG2E = 1.4426950408889634

_RB = 512
_QB = 512


def _proj_kernel(xq_ref, xk_ref, xv_ref, wq_ref, wk_ref, wv_ref,
                 q_out, k_out, vt_out):
    c = jnp.float32(0.125 * _LOG2E)
    accq = jnp.dot(xq_ref[...].astype(jnp.bfloat16), wq_ref[...],
                   preferred_element_type=jnp.float32)
    q_out[...] = (accq * c).astype(jnp.bfloat16)
    acck = jnp.dot(xk_ref[...].astype(jnp.bfloat16), wk_ref[...],
                   preferred_element_type=jnp.float32)
    k_out[...] = acck.astype(jnp.bfloat16)
    accv = jnp.dot(xv_ref[...].astype(jnp.bfloat16), wv_ref[...],
                   preferred_element_type=jnp.float32)
    vt_out[0] = accv.T.astype(jnp.bfloat16)


def _attn_kernel(valid_ref, q_ref, k_ref, vt_ref, o_ref):
    b = pl.program_id(0)
    h2 = pl.program_id(1)
    riota = lax.broadcasted_iota(jnp.int32, (_S, _QB), 0)
    outs = []
    for j in range(2):
        sl = slice(_HD * j, _HD * (j + 1))
        q = q_ref[0, :, sl]
        k = k_ref[0, :, sl]
        vt = vt_ref[0, sl, :]
        sT = lax.dot_general(k, q, (((1,), (1,)), ((), ())),
                             preferred_element_type=jnp.float32)
        valid = valid_ref[(b * _H + 2 * h2 + j) % _B]
        sT = sT + jnp.where(riota >= valid, jnp.float32(_MASK),
                            jnp.float32(0.0))
        m = jnp.max(sT, axis=0, keepdims=True)
        e = jnp.exp2(sT - m)
        l = jnp.sum(e, axis=0, keepdims=True)
        oT = lax.dot_general(vt, e.astype(jnp.bfloat16),
                             (((1,), (0,)), ((), ())),
                             preferred_element_type=jnp.float32)
        oT = oT * (1.0 / l)
        outs.append(oT.T.astype(jnp.bfloat16))
    o_ref[0] = jnp.concatenate(outs, axis=1)


def _oproj_kernel(x_ref, wo_ref, o_ref):
    o_ref[...] = jnp.dot(x_ref[...], wo_ref[...],
                         preferred_element_type=jnp.float32)


def kernel(query, key, value, valid_length, Wq, Wk, Wv, Wo):
    xq = query.reshape(_B * _S, _D)
    xk = key.reshape(_B * _S, _D)
    xv = value.reshape(_B * _S, _D)
    wq = Wq.astype(jnp.bfloat16)
    wk = Wk.astype(jnp.bfloat16)
    wv = Wv.astype(jnp.bfloat16)
    wo = Wo.astype(jnp.bfloat16)

    n_rows = _B * _S
    nrb = n_rows // _RB
    sb = _S // _RB
    row_spec = pl.BlockSpec((_RB, _D), lambda i: (i, 0))
    w_spec = pl.BlockSpec((_D, _D), lambda i: (0, 0))
    qp, kp, vt = pl.pallas_call(
        _proj_kernel,
        grid=(nrb,),
        in_specs=[row_spec, row_spec, row_spec, w_spec, w_spec, w_spec],
        out_specs=[
            row_spec,
            row_spec,
            pl.BlockSpec((1, _D, _RB), lambda i: (i // sb, 0, i % sb)),
        ],
        out_shape=[
            jax.ShapeDtypeStruct((n_rows, _D), jnp.bfloat16),
            jax.ShapeDtypeStruct((n_rows, _D), jnp.bfloat16),
            jax.ShapeDtypeStruct((_B, _D, _S), jnp.bfloat16),
        ],
        compiler_params=pltpu.CompilerParams(
            dimension_semantics=("parallel",),
            vmem_limit_bytes=64 * 1024 * 1024,
        ),
    )(xq, xk, xv, wq, wk, wv)

    qp = qp.reshape(_B, _S, _D)
    kp = kp.reshape(_B, _S, _D)

    attn = pl.pallas_call(
        _attn_kernel,
        grid=(_B, _H // 2, _S // _QB),
        in_specs=[
            pl.BlockSpec(memory_space=pltpu.SMEM),
            pl.BlockSpec((1, _QB, 2 * _HD), lambda b, h2, qi: (b, qi, h2)),
            pl.BlockSpec((1, _S, 2 * _HD), lambda b, h2, qi: (b, 0, h2)),
            pl.BlockSpec((1, 2 * _HD, _S), lambda b, h2, qi: (b, h2, 0)),
        ],
        out_specs=pl.BlockSpec((1, _QB, 2 * _HD),
                               lambda b, h2, qi: (b, qi, h2)),
        out_shape=jax.ShapeDtypeStruct((_B, _S, _D), jnp.bfloat16),
        compiler_params=pltpu.CompilerParams(
            dimension_semantics=("parallel", "arbitrary", "arbitrary"),
            vmem_limit_bytes=64 * 1024 * 1024,
        ),
    )(valid_length, qp, kp, vt)

    out = pl.pallas_call(
        _oproj_kernel,
        grid=(nrb,),
        in_specs=[row_spec, w_spec],
        out_specs=row_spec,
        out_shape=jax.ShapeDtypeStruct((n_rows, _D), jnp.float32),
        compiler_params=pltpu.CompilerParams(
            dimension_semantics=("parallel",),
            vmem_limit_bytes=64 * 1024 * 1024,
        ),
    )(attn.reshape(n_rows, _D), wo)
    return out.reshape(_B, _S, _D)


# speedup vs baseline: 1.0008x; 1.0008x over previous
import jax
import jax.numpy as jnp
from jax import lax
from jax.experimental import pallas as pl
from jax.experimental.pallas import tpu as pltpu

_B, _S, _D = 4, 2048, 1024
_H = 16
_HD = _D // _H
_MASK = -1000000.0
_LOG2E = 1.4426950408889634

_RB = 512
_QB = 512


def _proj_kernel(xq_ref, xk_ref, xv_ref, wq_ref, wk_ref, wv_ref,
                 q_out, k_out, vt_out):
    c = jnp.float32(0.125 * _LOG2E)
    accq = jnp.dot(xq_ref[...].astype(jnp.bfloat16), wq_ref[...],
                   preferred_element_type=jnp.float32)
    q_out[...] = (accq * c).astype(jnp.bfloat16)
    acck = jnp.dot(xk_ref[...].astype(jnp.bfloat16), wk_ref[...],
                   preferred_element_type=jnp.float32)
    k_out[...] = acck.astype(jnp.bfloat16)
    accv = jnp.dot(xv_ref[...].astype(jnp.bfloat16), wv_ref[...],
                   preferred_element_type=jnp.float32)
    vt_out[0] = accv.T.astype(jnp.bfloat16)


def _attn_kernel(valid_ref, q_ref, k_ref, vt_ref, o_ref):
    bh = pl.program_id(0)
    b = bh // (_H // 2)
    h2 = bh % (_H // 2)
    riota = lax.broadcasted_iota(jnp.int32, (_S, _QB), 0)
    outs = []
    for j in range(2):
        sl = slice(_HD * j, _HD * (j + 1))
        q = q_ref[0, :, sl]
        k = k_ref[0, :, sl]
        vt = vt_ref[0, sl, :]
        sT = lax.dot_general(k, q, (((1,), (1,)), ((), ())),
                             preferred_element_type=jnp.float32)
        valid = valid_ref[(b * _H + 2 * h2 + j) % _B]
        sT = sT + jnp.where(riota >= valid, jnp.float32(_MASK),
                            jnp.float32(0.0))
        m = jnp.max(sT, axis=0, keepdims=True)
        e = jnp.exp2(sT - m)
        l = jnp.sum(e, axis=0, keepdims=True)
        oT = lax.dot_general(vt, e.astype(jnp.bfloat16),
                             (((1,), (0,)), ((), ())),
                             preferred_element_type=jnp.float32)
        oT = oT * (1.0 / l)
        outs.append(oT.T.astype(jnp.bfloat16))
    o_ref[0] = jnp.concatenate(outs, axis=1)


def _oproj_kernel(x_ref, wo_ref, o_ref):
    o_ref[...] = jnp.dot(x_ref[...], wo_ref[...],
                         preferred_element_type=jnp.float32)


def kernel(query, key, value, valid_length, Wq, Wk, Wv, Wo):
    xq = query.reshape(_B * _S, _D)
    xk = key.reshape(_B * _S, _D)
    xv = value.reshape(_B * _S, _D)
    wq = Wq.astype(jnp.bfloat16)
    wk = Wk.astype(jnp.bfloat16)
    wv = Wv.astype(jnp.bfloat16)
    wo = Wo.astype(jnp.bfloat16)

    n_rows = _B * _S
    nrb = n_rows // _RB
    sb = _S // _RB
    row_spec = pl.BlockSpec((_RB, _D), lambda i: (i, 0))
    w_spec = pl.BlockSpec((_D, _D), lambda i: (0, 0))
    qp, kp, vt = pl.pallas_call(
        _proj_kernel,
        grid=(nrb,),
        in_specs=[row_spec, row_spec, row_spec, w_spec, w_spec, w_spec],
        out_specs=[
            row_spec,
            row_spec,
            pl.BlockSpec((1, _D, _RB), lambda i: (i // sb, 0, i % sb)),
        ],
        out_shape=[
            jax.ShapeDtypeStruct((n_rows, _D), jnp.bfloat16),
            jax.ShapeDtypeStruct((n_rows, _D), jnp.bfloat16),
            jax.ShapeDtypeStruct((_B, _D, _S), jnp.bfloat16),
        ],
        compiler_params=pltpu.CompilerParams(
            dimension_semantics=("parallel",),
            vmem_limit_bytes=64 * 1024 * 1024,
        ),
    )(xq, xk, xv, wq, wk, wv)

    qp = qp.reshape(_B, _S, _D)
    kp = kp.reshape(_B, _S, _D)

    attn = pl.pallas_call(
        _attn_kernel,
        grid=(_B * _H // 2, _S // _QB),
        in_specs=[
            pl.BlockSpec(memory_space=pltpu.SMEM),
            pl.BlockSpec((1, _QB, 2 * _HD),
                         lambda bh, qi: (bh // (_H // 2), qi, bh % (_H // 2))),
            pl.BlockSpec((1, _S, 2 * _HD),
                         lambda bh, qi: (bh // (_H // 2), 0, bh % (_H // 2))),
            pl.BlockSpec((1, 2 * _HD, _S),
                         lambda bh, qi: (bh // (_H // 2), bh % (_H // 2), 0)),
        ],
        out_specs=pl.BlockSpec((1, _QB, 2 * _HD),
                               lambda bh, qi: (bh // (_H // 2), qi,
                                               bh % (_H // 2))),
        out_shape=jax.ShapeDtypeStruct((_B, _S, _D), jnp.bfloat16),
        compiler_params=pltpu.CompilerParams(
            dimension_semantics=("parallel", "arbitrary"),
            vmem_limit_bytes=64 * 1024 * 1024,
        ),
    )(valid_length, qp, kp, vt)

    out = pl.pallas_call(
        _oproj_kernel,
        grid=(nrb,),
        in_specs=[row_spec, w_spec],
        out_specs=row_spec,
        out_shape=jax.ShapeDtypeStruct((n_rows, _D), jnp.float32),
        compiler_params=pltpu.CompilerParams(
            dimension_semantics=("parallel",),
            vmem_limit_bytes=64 * 1024 * 1024,
        ),
    )(attn.reshape(n_rows, _D), wo)
    return out.reshape(_B, _S, _D)


# speedup vs baseline: 1.0320x; 1.0311x over previous
import jax
import jax.numpy as jnp
from jax import lax
from jax.experimental import pallas as pl
from jax.experimental.pallas import tpu as pltpu

_B, _S, _D = 4, 2048, 1024
_H = 16
_HD = _D // _H
_MASK = -1000000.0
_LOG2E = 1.4426950408889634

_RB = 512
_QB = 512
_CK = 256


def _proj_kernel(xq_ref, xk_ref, xv_ref, wq_ref, wk_ref, wv_ref,
                 q_out, k_out, vt_out):
    c = jnp.float32(0.125 * _LOG2E)
    accq = jnp.dot(xq_ref[...].astype(jnp.bfloat16),
                   wq_ref[...].astype(jnp.bfloat16),
                   preferred_element_type=jnp.float32)
    q_out[...] = (accq * c).astype(jnp.bfloat16)
    acck = jnp.dot(xk_ref[...].astype(jnp.bfloat16),
                   wk_ref[...].astype(jnp.bfloat16),
                   preferred_element_type=jnp.float32)
    k_out[...] = acck.astype(jnp.bfloat16)
    accv = jnp.dot(xv_ref[...].astype(jnp.bfloat16),
                   wv_ref[...].astype(jnp.bfloat16),
                   preferred_element_type=jnp.float32)
    vt_out[0] = accv.T.astype(jnp.bfloat16)


def _attn_kernel(valid_ref, q_ref, k_ref, vt_ref, o_ref, s_ref):
    bh = pl.program_id(0)
    b = bh // (_H // 2)
    h2 = bh % (_H // 2)
    riota = lax.broadcasted_iota(jnp.int32, (_S, _QB), 0)
    outs = []
    for j in range(2):
        sl = slice(_HD * j, _HD * (j + 1))
        q = q_ref[0, :, sl]
        k = k_ref[0, :, sl]
        valid = valid_ref[(b * _H + 2 * h2 + j) % _B]
        s_ref[j] = lax.dot_general(
            k, q, (((1,), (1,)), ((), ())),
            preferred_element_type=jnp.float32,
        ) + jnp.where(riota >= valid, jnp.float32(_MASK), jnp.float32(0.0))
        m = jnp.max(s_ref[j], axis=0, keepdims=True)
        l = jnp.zeros((1, _QB), jnp.float32)
        oT = jnp.zeros((_HD, _QB), jnp.float32)
        for ci in range(_S // _CK):
            ch = s_ref[j, ci * _CK:(ci + 1) * _CK, :]
            e = jnp.exp2(ch - m)
            l = l + jnp.sum(e, axis=0, keepdims=True)
            oT = oT + lax.dot_general(
                vt_ref[0, sl, ci * _CK:(ci + 1) * _CK],
                e.astype(jnp.bfloat16),
                (((1,), (0,)), ((), ())),
                preferred_element_type=jnp.float32)
        oT = oT * (1.0 / l)
        outs.append(oT.T.astype(jnp.bfloat16))
    o_ref[0] = jnp.concatenate(outs, axis=1)


def _oproj_kernel(x_ref, wo_ref, o_ref):
    o_ref[...] = jnp.dot(x_ref[...], wo_ref[...].astype(jnp.bfloat16),
                         preferred_element_type=jnp.float32)


def kernel(query, key, value, valid_length, Wq, Wk, Wv, Wo):
    xq = query.reshape(_B * _S, _D)
    xk = key.reshape(_B * _S, _D)
    xv = value.reshape(_B * _S, _D)

    n_rows = _B * _S
    nrb = n_rows // _RB
    sb = _S // _RB
    row_spec = pl.BlockSpec((_RB, _D), lambda i: (i, 0))
    w_spec = pl.BlockSpec((_D, _D), lambda i: (0, 0))
    qp, kp, vt = pl.pallas_call(
        _proj_kernel,
        grid=(nrb,),
        in_specs=[row_spec, row_spec, row_spec, w_spec, w_spec, w_spec],
        out_specs=[
            row_spec,
            row_spec,
            pl.BlockSpec((1, _D, _RB), lambda i: (i // sb, 0, i % sb)),
        ],
        out_shape=[
            jax.ShapeDtypeStruct((n_rows, _D), jnp.bfloat16),
            jax.ShapeDtypeStruct((n_rows, _D), jnp.bfloat16),
            jax.ShapeDtypeStruct((_B, _D, _S), jnp.bfloat16),
        ],
        compiler_params=pltpu.CompilerParams(
            dimension_semantics=("parallel",),
            vmem_limit_bytes=64 * 1024 * 1024,
        ),
    )(xq, xk, xv, Wq, Wk, Wv)

    qp = qp.reshape(_B, _S, _D)
    kp = kp.reshape(_B, _S, _D)

    nh2 = _H // 2
    attn = pl.pallas_call(
        _attn_kernel,
        grid=(_B * nh2, _S // _QB),
        in_specs=[
            pl.BlockSpec(memory_space=pltpu.SMEM),
            pl.BlockSpec((1, _QB, 2 * _HD),
                         lambda bh, qi: (bh // nh2, qi, bh % nh2)),
            pl.BlockSpec((1, _S, 2 * _HD),
                         lambda bh, qi: (bh // nh2, 0, bh % nh2)),
            pl.BlockSpec((1, 2 * _HD, _S),
                         lambda bh, qi: (bh // nh2, bh % nh2, 0)),
        ],
        out_specs=pl.BlockSpec((1, _QB, 2 * _HD),
                               lambda bh, qi: (bh // nh2, qi, bh % nh2)),
        out_shape=jax.ShapeDtypeStruct((_B, _S, _D), jnp.bfloat16),
        scratch_shapes=[pltpu.VMEM((2, _S, _QB), jnp.float32)],
        compiler_params=pltpu.CompilerParams(
            dimension_semantics=("parallel", "arbitrary"),
            vmem_limit_bytes=64 * 1024 * 1024,
        ),
    )(valid_length, qp, kp, vt)

    out = pl.pallas_call(
        _oproj_kernel,
        grid=(nrb,),
        in_specs=[row_spec, w_spec],
        out_specs=row_spec,
        out_shape=jax.ShapeDtypeStruct((n_rows, _D), jnp.float32),
        compiler_params=pltpu.CompilerParams(
            dimension_semantics=("parallel",),
            vmem_limit_bytes=64 * 1024 * 1024,
        ),
    )(attn.reshape(n_rows, _D), Wo)
    return out.reshape(_B, _S, _D)


# speedup vs baseline: 1.0489x; 1.0164x over previous
import jax
import jax.numpy as jnp
from jax import lax
from jax.experimental import pallas as pl
from jax.experimental.pallas import tpu as pltpu

_B, _S, _D = 4, 2048, 1024
_H = 16
_HD = _D // _H
_MASK = -1000000.0
_LOG2E = 1.4426950408889634

_RB = 512
_QB = 512
_CK = 256


def _proj_kernel(xq_ref, xk_ref, xv_ref, wq_ref, wk_ref, wv_ref,
                 q_out, k_out, vt_out):
    c = jnp.float32(0.125 * _LOG2E)
    accq = jnp.dot(xq_ref[...].astype(jnp.bfloat16),
                   wq_ref[...].astype(jnp.bfloat16),
                   preferred_element_type=jnp.float32)
    q_out[...] = (accq * c).astype(jnp.bfloat16)
    acck = jnp.dot(xk_ref[...].astype(jnp.bfloat16),
                   wk_ref[...].astype(jnp.bfloat16),
                   preferred_element_type=jnp.float32)
    k_out[...] = acck.astype(jnp.bfloat16)
    accv = jnp.dot(xv_ref[...].astype(jnp.bfloat16),
                   wv_ref[...].astype(jnp.bfloat16),
                   preferred_element_type=jnp.float32)
    vt_out[0] = accv.T.astype(jnp.bfloat16)


def _attn_kernel(valid_ref, q_ref, k_ref, vt_ref, o_ref, s_ref):
    bh = pl.program_id(0)
    b = bh // (_H // 2)
    h2 = bh % (_H // 2)
    liota = lax.broadcasted_iota(jnp.int32, (1, _S), 1)
    outs = []
    for j in range(2):
        sl = slice(_HD * j, _HD * (j + 1))
        q = q_ref[0, :, sl]
        k = k_ref[0, :, sl]
        valid = valid_ref[(b * _H + 2 * h2 + j) % _B]
        s_ref[j] = lax.dot_general(
            k, q, (((1,), (1,)), ((), ())),
            preferred_element_type=jnp.float32)
        m = jnp.max(s_ref[j], axis=0, keepdims=True)
        maskv = jnp.where(liota >= valid, jnp.float32(0.0),
                          jnp.float32(1.0)).astype(jnp.bfloat16)
        vt_aug = jnp.concatenate(
            [vt_ref[0, sl, :] * maskv,
             jnp.broadcast_to(maskv, (16, _S))], axis=0)
        oT = jnp.zeros((_HD + 16, _QB), jnp.float32)
        for ci in range(_S // _CK):
            ch = s_ref[j, ci * _CK:(ci + 1) * _CK, :]
            e = jnp.exp2(ch - m)
            oT = oT + lax.dot_general(
                vt_aug[:, ci * _CK:(ci + 1) * _CK],
                e.astype(jnp.bfloat16),
                (((1,), (0,)), ((), ())),
                preferred_element_type=jnp.float32)
        l = oT[_HD:_HD + 1, :]
        o = oT[:_HD, :] * (1.0 / l)
        outs.append(o.T.astype(jnp.bfloat16))
    o_ref[0] = jnp.concatenate(outs, axis=1)


def _oproj_kernel(x_ref, wo_ref, o_ref):
    o_ref[...] = jnp.dot(x_ref[...], wo_ref[...].astype(jnp.bfloat16),
                         preferred_element_type=jnp.float32)


def kernel(query, key, value, valid_length, Wq, Wk, Wv, Wo):
    xq = query.reshape(_B * _S, _D)
    xk = key.reshape(_B * _S, _D)
    xv = value.reshape(_B * _S, _D)

    n_rows = _B * _S
    nrb = n_rows // _RB
    sb = _S // _RB
    row_spec = pl.BlockSpec((_RB, _D), lambda i: (i, 0))
    w_spec = pl.BlockSpec((_D, _D), lambda i: (0, 0))
    qp, kp, vt = pl.pallas_call(
        _proj_kernel,
        grid=(nrb,),
        in_specs=[row_spec, row_spec, row_spec, w_spec, w_spec, w_spec],
        out_specs=[
            row_spec,
            row_spec,
            pl.BlockSpec((1, _D, _RB), lambda i: (i // sb, 0, i % sb)),
        ],
        out_shape=[
            jax.ShapeDtypeStruct((n_rows, _D), jnp.bfloat16),
            jax.ShapeDtypeStruct((n_rows, _D), jnp.bfloat16),
            jax.ShapeDtypeStruct((_B, _D, _S), jnp.bfloat16),
        ],
        compiler_params=pltpu.CompilerParams(
            dimension_semantics=("parallel",),
            vmem_limit_bytes=64 * 1024 * 1024,
        ),
    )(xq, xk, xv, Wq, Wk, Wv)

    qp = qp.reshape(_B, _S, _D)
    kp = kp.reshape(_B, _S, _D)

    nh2 = _H // 2
    attn = pl.pallas_call(
        _attn_kernel,
        grid=(_B * nh2, _S // _QB),
        in_specs=[
            pl.BlockSpec(memory_space=pltpu.SMEM),
            pl.BlockSpec((1, _QB, 2 * _HD),
                         lambda bh, qi: (bh // nh2, qi, bh % nh2)),
            pl.BlockSpec((1, _S, 2 * _HD),
                         lambda bh, qi: (bh // nh2, 0, bh % nh2)),
            pl.BlockSpec((1, 2 * _HD, _S),
                         lambda bh, qi: (bh // nh2, bh % nh2, 0)),
        ],
        out_specs=pl.BlockSpec((1, _QB, 2 * _HD),
                               lambda bh, qi: (bh // nh2, qi, bh % nh2)),
        out_shape=jax.ShapeDtypeStruct((_B, _S, _D), jnp.bfloat16),
        scratch_shapes=[pltpu.VMEM((2, _S, _QB), jnp.float32)],
        compiler_params=pltpu.CompilerParams(
            dimension_semantics=("parallel", "arbitrary"),
            vmem_limit_bytes=64 * 1024 * 1024,
        ),
    )(valid_length, qp, kp, vt)

    out = pl.pallas_call(
        _oproj_kernel,
        grid=(nrb,),
        in_specs=[row_spec, w_spec],
        out_specs=row_spec,
        out_shape=jax.ShapeDtypeStruct((n_rows, _D), jnp.float32),
        compiler_params=pltpu.CompilerParams(
            dimension_semantics=("parallel",),
            vmem_limit_bytes=64 * 1024 * 1024,
        ),
    )(attn.reshape(n_rows, _D), Wo)
    return out.reshape(_B, _S, _D)


# speedup vs baseline: 1.1143x; 1.0623x over previous
import jax
import jax.numpy as jnp
from jax import lax
from jax.experimental import pallas as pl
from jax.experimental.pallas import tpu as pltpu

_B, _S, _D = 4, 2048, 1024
_H = 16
_HD = _D // _H
_MASK = -1000000.0
_LOG2E = 1.4426950408889634

_RB = 512
_QB = 512
_CK = 256


def _proj_kernel(xq_ref, xk_ref, xv_ref, wq_ref, wk_ref, wv_ref,
                 q_out, k_out, vt_out):
    c = jnp.float32(0.125 * _LOG2E)
    accq = jnp.dot(xq_ref[...].astype(jnp.bfloat16),
                   wq_ref[...].astype(jnp.bfloat16),
                   preferred_element_type=jnp.float32)
    q_out[...] = (accq * c).astype(jnp.bfloat16)
    acck = jnp.dot(xk_ref[...].astype(jnp.bfloat16),
                   wk_ref[...].astype(jnp.bfloat16),
                   preferred_element_type=jnp.float32)
    k_out[...] = acck.astype(jnp.bfloat16)
    accv = jnp.dot(xv_ref[...].astype(jnp.bfloat16),
                   wv_ref[...].astype(jnp.bfloat16),
                   preferred_element_type=jnp.float32)
    vt_out[0] = accv.T.astype(jnp.bfloat16)


def _attn_kernel(valid_ref, q_ref, k_ref, vt_ref, o_ref, s_ref):
    g = pl.program_id(0)
    b = g // 2
    hg = g % 2
    liota = lax.broadcasted_iota(jnp.int32, (1, _S), 1)
    outs = []
    for j in range(_H // 2):
        sl = slice(_HD * j, _HD * (j + 1))
        q = q_ref[0, :, sl]
        k = k_ref[0, :, sl]
        valid = valid_ref[(b * _H + (_H // 2) * hg + j) % _B]
        s_ref[j % 2] = lax.dot_general(
            k, q, (((1,), (1,)), ((), ())),
            preferred_element_type=jnp.float32)
        m = jnp.max(s_ref[j % 2], axis=0, keepdims=True)
        maskv = jnp.where(liota >= valid, jnp.float32(0.0),
                          jnp.float32(1.0)).astype(jnp.bfloat16)
        vt_aug = jnp.concatenate(
            [vt_ref[0, sl, :] * maskv,
             jnp.broadcast_to(maskv, (16, _S))], axis=0)
        oT = jnp.zeros((_HD + 16, _QB), jnp.float32)
        for ci in range(_S // _CK):
            ch = s_ref[j % 2, ci * _CK:(ci + 1) * _CK, :]
            e = jnp.exp2(ch - m)
            oT = oT + lax.dot_general(
                vt_aug[:, ci * _CK:(ci + 1) * _CK],
                e.astype(jnp.bfloat16),
                (((1,), (0,)), ((), ())),
                preferred_element_type=jnp.float32)
        l = oT[_HD:_HD + 1, :]
        o = oT[:_HD, :] * (1.0 / l)
        outs.append(o.T.astype(jnp.bfloat16))
    o_ref[0] = jnp.concatenate(outs, axis=1)


def _oproj_kernel(x_ref, wo_ref, o_ref):
    o_ref[...] = jnp.dot(x_ref[...], wo_ref[...].astype(jnp.bfloat16),
                         preferred_element_type=jnp.float32)


def kernel(query, key, value, valid_length, Wq, Wk, Wv, Wo):
    xq = query.reshape(_B * _S, _D)
    xk = key.reshape(_B * _S, _D)
    xv = value.reshape(_B * _S, _D)

    n_rows = _B * _S
    nrb = n_rows // _RB
    sb = _S // _RB
    row_spec = pl.BlockSpec((_RB, _D), lambda i: (i, 0))
    w_spec = pl.BlockSpec((_D, _D), lambda i: (0, 0))
    qp, kp, vt = pl.pallas_call(
        _proj_kernel,
        grid=(nrb,),
        in_specs=[row_spec, row_spec, row_spec, w_spec, w_spec, w_spec],
        out_specs=[
            row_spec,
            row_spec,
            pl.BlockSpec((1, _D, _RB), lambda i: (i // sb, 0, i % sb)),
        ],
        out_shape=[
            jax.ShapeDtypeStruct((n_rows, _D), jnp.bfloat16),
            jax.ShapeDtypeStruct((n_rows, _D), jnp.bfloat16),
            jax.ShapeDtypeStruct((_B, _D, _S), jnp.bfloat16),
        ],
        compiler_params=pltpu.CompilerParams(
            dimension_semantics=("parallel",),
            vmem_limit_bytes=64 * 1024 * 1024,
        ),
    )(xq, xk, xv, Wq, Wk, Wv)

    qp = qp.reshape(_B, _S, _D)
    kp = kp.reshape(_B, _S, _D)

    hw = _D // 2
    attn = pl.pallas_call(
        _attn_kernel,
        grid=(_B * 2, _S // _QB),
        in_specs=[
            pl.BlockSpec(memory_space=pltpu.SMEM),
            pl.BlockSpec((1, _QB, hw), lambda g, qi: (g // 2, qi, g % 2)),
            pl.BlockSpec((1, _S, hw), lambda g, qi: (g // 2, 0, g % 2)),
            pl.BlockSpec((1, hw, _S), lambda g, qi: (g // 2, g % 2, 0)),
        ],
        out_specs=pl.BlockSpec((1, _QB, hw),
                               lambda g, qi: (g // 2, qi, g % 2)),
        out_shape=jax.ShapeDtypeStruct((_B, _S, _D), jnp.bfloat16),
        scratch_shapes=[pltpu.VMEM((2, _S, _QB), jnp.float32)],
        compiler_params=pltpu.CompilerParams(
            dimension_semantics=("parallel", "arbitrary"),
            vmem_limit_bytes=64 * 1024 * 1024,
        ),
    )(valid_length, qp, kp, vt)

    out = pl.pallas_call(
        _oproj_kernel,
        grid=(nrb,),
        in_specs=[row_spec, w_spec],
        out_specs=row_spec,
        out_shape=jax.ShapeDtypeStruct((n_rows, _D), jnp.float32),
        compiler_params=pltpu.CompilerParams(
            dimension_semantics=("parallel",),
            vmem_limit_bytes=64 * 1024 * 1024,
        ),
    )(attn.reshape(n_rows, _D), Wo)
    return out.reshape(_B, _S, _D)


# speedup vs baseline: 1.2118x; 1.0875x over previous
import jax
import jax.numpy as jnp
from jax import lax
from jax.experimental import pallas as pl
from jax.experimental.pallas import tpu as pltpu

_B, _S, _D = 4, 2048, 1024
_H = 16
_HD = _D // _H
_MASK = -1000000.0
_LOG2E = 1.4426950408889634

_RB = 512
_QB = 512
_CK = 256


def _proj_kernel(xq_ref, xk_ref, xv_ref, wq_ref, wk_ref, wv_ref,
                 q_out, k_out, vt_out):
    c = jnp.float32(0.125 * _LOG2E)
    accq = jnp.dot(xq_ref[...].astype(jnp.bfloat16),
                   wq_ref[...].astype(jnp.bfloat16),
                   preferred_element_type=jnp.float32)
    q_out[...] = (accq * c).astype(jnp.bfloat16)
    acck = jnp.dot(xk_ref[...].astype(jnp.bfloat16),
                   wk_ref[...].astype(jnp.bfloat16),
                   preferred_element_type=jnp.float32)
    k_out[...] = acck.astype(jnp.bfloat16)
    accv = jnp.dot(xv_ref[...].astype(jnp.bfloat16),
                   wv_ref[...].astype(jnp.bfloat16),
                   preferred_element_type=jnp.float32)
    vt_out[0] = accv.T.astype(jnp.bfloat16)


def _attn_kernel(valid_ref, q_ref, k_ref, vt_ref, o_ref, s_ref):
    g = pl.program_id(0)
    b = g // 2
    hg = g % 2
    liota = lax.broadcasted_iota(jnp.int32, (1, _S), 1)
    outs = []
    for j in range(_H // 2):
        sl = slice(_HD * j, _HD * (j + 1))
        q = q_ref[0, :, sl]
        k = k_ref[0, :, sl]
        valid = valid_ref[(b * _H + (_H // 2) * hg + j) % _B]
        s_ref[j % 2] = lax.dot_general(
            k, q, (((1,), (1,)), ((), ())),
            preferred_element_type=jnp.float32)
        m = jnp.max(s_ref[j % 2], axis=0, keepdims=True)
        maskv = jnp.where(liota >= valid, jnp.float32(0.0),
                          jnp.float32(1.0)).astype(jnp.bfloat16)
        vt_aug = jnp.concatenate(
            [vt_ref[0, sl, :] * maskv,
             jnp.broadcast_to(maskv, (16, _S))], axis=0)
        oT = jnp.zeros((_HD + 16, _QB), jnp.float32)
        mb = m.astype(jnp.bfloat16)
        for ci in range(_S // _CK):
            ch = s_ref[j % 2, ci * _CK:(ci + 1) * _CK, :]
            e = jnp.exp2(ch.astype(jnp.bfloat16) - mb)
            oT = oT + lax.dot_general(
                vt_aug[:, ci * _CK:(ci + 1) * _CK], e,
                (((1,), (0,)), ((), ())),
                preferred_element_type=jnp.float32)
        l = oT[_HD:_HD + 1, :]
        o = oT[:_HD, :] * (1.0 / l)
        outs.append(o.T.astype(jnp.bfloat16))
    o_ref[0] = jnp.concatenate(outs, axis=1)


def _oproj_kernel(x_ref, wo_ref, o_ref):
    o_ref[...] = jnp.dot(x_ref[...], wo_ref[...].astype(jnp.bfloat16),
                         preferred_element_type=jnp.float32)


def kernel(query, key, value, valid_length, Wq, Wk, Wv, Wo):
    xq = query.reshape(_B * _S, _D)
    xk = key.reshape(_B * _S, _D)
    xv = value.reshape(_B * _S, _D)

    n_rows = _B * _S
    nrb = n_rows // _RB
    sb = _S // _RB
    row_spec = pl.BlockSpec((_RB, _D), lambda i: (i, 0))
    w_spec = pl.BlockSpec((_D, _D), lambda i: (0, 0))
    qp, kp, vt = pl.pallas_call(
        _proj_kernel,
        grid=(nrb,),
        in_specs=[row_spec, row_spec, row_spec, w_spec, w_spec, w_spec],
        out_specs=[
            row_spec,
            row_spec,
            pl.BlockSpec((1, _D, _RB), lambda i: (i // sb, 0, i % sb)),
        ],
        out_shape=[
            jax.ShapeDtypeStruct((n_rows, _D), jnp.bfloat16),
            jax.ShapeDtypeStruct((n_rows, _D), jnp.bfloat16),
            jax.ShapeDtypeStruct((_B, _D, _S), jnp.bfloat16),
        ],
        compiler_params=pltpu.CompilerParams(
            dimension_semantics=("parallel",),
            vmem_limit_bytes=64 * 1024 * 1024,
        ),
    )(xq, xk, xv, Wq, Wk, Wv)

    qp = qp.reshape(_B, _S, _D)
    kp = kp.reshape(_B, _S, _D)

    hw = _D // 2
    attn = pl.pallas_call(
        _attn_kernel,
        grid=(_B * 2, _S // _QB),
        in_specs=[
            pl.BlockSpec(memory_space=pltpu.SMEM),
            pl.BlockSpec((1, _QB, hw), lambda g, qi: (g // 2, qi, g % 2)),
            pl.BlockSpec((1, _S, hw), lambda g, qi: (g // 2, 0, g % 2)),
            pl.BlockSpec((1, hw, _S), lambda g, qi: (g // 2, g % 2, 0)),
        ],
        out_specs=pl.BlockSpec((1, _QB, hw),
                               lambda g, qi: (g // 2, qi, g % 2)),
        out_shape=jax.ShapeDtypeStruct((_B, _S, _D), jnp.bfloat16),
        scratch_shapes=[pltpu.VMEM((2, _S, _QB), jnp.float32)],
        compiler_params=pltpu.CompilerParams(
            dimension_semantics=("parallel", "arbitrary"),
            vmem_limit_bytes=64 * 1024 * 1024,
        ),
    )(valid_length, qp, kp, vt)

    out = pl.pallas_call(
        _oproj_kernel,
        grid=(nrb,),
        in_specs=[row_spec, w_spec],
        out_specs=row_spec,
        out_shape=jax.ShapeDtypeStruct((n_rows, _D), jnp.float32),
        compiler_params=pltpu.CompilerParams(
            dimension_semantics=("parallel",),
            vmem_limit_bytes=64 * 1024 * 1024,
        ),
    )(attn.reshape(n_rows, _D), Wo)
    return out.reshape(_B, _S, _D)
